# baseline (speedup 1.0000x reference)
import functools

import jax
import jax.numpy as jnp
from jax import lax
from jax.experimental import pallas as pl
from jax.experimental.pallas import tpu as pltpu
from jax.experimental.pallas import tpu_sc as plsc

NC = 2
NS = 16
NW = NC * NS
LANES = 16
CHUNK = 128
H = 64
DEPTH = 8
NROW = 1_000_000
PB = 1 << 17
NJ = 8
HP = 32
J_STRIDE = 4 * PB
A_STRIDE = NJ * J_STRIDE
FLAT = 8 * A_STRIDE


def _tc_detile(wt3):
    def body(in_ref, out_ref):
        x = in_ref[0].reshape(4, 2, PB)
        evb = lax.bitcast_convert_type(x[:, 0, :], jnp.uint32)
        odb = lax.bitcast_convert_type(x[:, 1, :], jnp.uint32)
        ev16 = (evb + 0x7FFF + ((evb >> 16) & 1)) >> 16
        od16 = (odb + 0x7FFF + ((odb >> 16) & 1)) >> 16
        packed = lax.bitcast_convert_type(ev16 | (od16 << 16), jnp.float32)
        out_ref[...] = packed.reshape(J_STRIDE // 128, 128)

    return pl.pallas_call(
        body,
        grid=(8, NJ),
        in_specs=[pl.BlockSpec((1, 8, PB), lambda a, j: (a, 0, j))],
        out_specs=pl.BlockSpec(
            (J_STRIDE // 128, 128), lambda a, j: (a * NJ + j, 0)),
        out_shape=jax.ShapeDtypeStruct((FLAT // 128, 128), jnp.float32),
    )(wt3)


def _build_sc(B):
    bpw = B // NW
    nch = bpw // CHUNK

    mesh = plsc.VectorSubcoreMesh(core_axis_name="c", subcore_axis_name="s")

    @functools.partial(
        pl.kernel,
        out_type=jax.ShapeDtypeStruct((B,), jnp.float32),
        mesh=mesh,
        compiler_params=pltpu.CompilerParams(
            needs_layout_passes=False, use_tc_tiling_on_sc=False),
        scratch_types=[
            pltpu.VMEM((nch, CHUNK), jnp.int32),
            pltpu.VMEM((nch, CHUNK), jnp.int32),
            pltpu.VMEM((HP, CHUNK), jnp.int32),
            pltpu.VMEM((HP, CHUNK), jnp.int32),
            pltpu.VMEM((HP * CHUNK,), jnp.float32),
            pltpu.VMEM((HP * CHUNK,), jnp.float32),
            pltpu.VMEM((bpw,), jnp.float32),
            pltpu.VMEM((bpw,), jnp.float32),
            pltpu.VMEM((bpw,), jnp.float32),
            pltpu.VMEM((LANES,), jnp.float32),
            pltpu.SemaphoreType.DMA,
        ],
    )
    def mf(user_hbm, item_hbm, uw_hbm, iw_hbm, ub_hbm, ib_hbm, bias_hbm,
           out_hbm, uid_v, iid_v, uhx_v, ihx_v, uval_v, ival_v,
           ubr_v, ibr_v, out_v, bias_v, sem):
        wid = lax.axis_index("s") * NC + lax.axis_index("c")
        base = wid * bpw

        pltpu.sync_copy(user_hbm.at[wid], uid_v)
        pltpu.sync_copy(item_hbm.at[wid], iid_v)
        pltpu.sync_copy(bias_hbm, bias_v)

        bias_copies = []
        for c in range(nch):
            sl = pl.ds(c * CHUNK, CHUNK)
            bias_copies.append(pltpu.make_async_copy(
                ub_hbm.at[uid_v.at[c]], ubr_v.at[sl], sem))
            bias_copies.append(pltpu.make_async_copy(
                ib_hbm.at[iid_v.at[c]], ibr_v.at[sl], sem))
        for cp in bias_copies:
            cp.start()

        bias_vec = bias_v[...]

        for c in range(nch):
            def build_body(hp, carry, _c=c):
                coff = (hp >> 2) * A_STRIDE + (hp & 3) * PB
                for g in range(CHUNK // LANES):
                    sl = pl.ds(g * LANES, LANES)
                    r = uid_v[_c, sl]
                    uhx_v[hp, sl] = (
                        (r >> 17) * J_STRIDE + (r & (PB - 1)) + coff)
                    r = iid_v[_c, sl]
                    ihx_v[hp, sl] = (
                        (r >> 17) * J_STRIDE + (r & (PB - 1)) + coff)
                return carry

            lax.fori_loop(0, HP, build_body, 0)

            def fire_one(hp):
                dst = pl.ds(hp * CHUNK, CHUNK)
                pltpu.make_async_copy(
                    uw_hbm.at[uhx_v.at[hp]], uval_v.at[dst], sem).start()
                pltpu.make_async_copy(
                    iw_hbm.at[ihx_v.at[hp]], ival_v.at[dst], sem).start()

            def wait_one(hp):
                dst = pl.ds(hp * CHUNK, CHUNK)
                pltpu.make_async_copy(
                    uw_hbm.at[uhx_v.at[hp]], uval_v.at[dst], sem).wait()
                pltpu.make_async_copy(
                    iw_hbm.at[ihx_v.at[hp]], ival_v.at[dst], sem).wait()

            def fire_body(hp, carry):
                fire_one(hp)

                @pl.when(hp >= DEPTH)
                def _():
                    wait_one(hp - DEPTH)
                return carry

            lax.fori_loop(0, HP, fire_body, 0)

            def drain_body(hp, carry):
                wait_one(hp)
                return carry

            lax.fori_loop(HP - DEPTH, HP, drain_body, 0)
            if c == 0:
                for cp in bias_copies:
                    cp.wait()

            himask = jnp.full((LANES,), 0xFFFF0000, jnp.uint32)

            def unpack2(w):
                bits = plsc.bitcast(w, jnp.uint32)
                ev = plsc.bitcast(bits << 16, jnp.float32)
                od = plsc.bitcast(bits & himask, jnp.float32)
                return ev, od

            for g in range(CHUNK // LANES):
                o = c * CHUNK + g * LANES
                ubv = ubr_v[pl.ds(o, LANES)]
                ibv = ibr_v[pl.ds(o, LANES)]

                def h_body(hp, acc, _g=g):
                    wu = uval_v[pl.ds(hp * CHUNK + _g * LANES, LANES)]
                    wi = ival_v[pl.ds(hp * CHUNK + _g * LANES, LANES)]
                    uev, uod = unpack2(wu)
                    iev, iod = unpack2(wi)
                    return (acc + (uev + ubv) * (iev + ibv)
                            + (uod + ubv) * (iod + ibv))

                acc = lax.fori_loop(
                    0, HP, h_body, jnp.zeros((LANES,), jnp.float32))
                out_v[pl.ds(o, LANES)] = acc + bias_vec

        pltpu.sync_copy(out_v, out_hbm.at[pl.ds(base, bpw)])

    return mf


def kernel(user, item, user_weight, item_weight, user_bias, item_bias, bias):
    B = user.shape[0]
    user_r = user.reshape(NW, B // NW // CHUNK, CHUNK)
    item_r = item.reshape(NW, B // NW // CHUNK, CHUNK)
    uw_flat = _tc_detile(user_weight.T.reshape(8, 8, NROW)).reshape(-1)
    iw_flat = _tc_detile(item_weight.T.reshape(8, 8, NROW)).reshape(-1)
    ub = user_bias.reshape(-1)
    ib = item_bias.reshape(-1)
    bias16 = jnp.broadcast_to(bias, (LANES,)).astype(jnp.float32)
    mf = _build_sc(B)
    return mf(user_r, item_r, uw_flat, iw_flat, ub, ib, bias16)


# speedup vs baseline: 3.0524x; 3.0524x over previous
import functools

import jax
import jax.numpy as jnp
from jax import lax
from jax.experimental import pallas as pl
from jax.experimental.pallas import tpu as pltpu
from jax.experimental.pallas import tpu_sc as plsc

NC = 2
NS = 16
NW = NC * NS
LANES = 16
CHUNK = 128
H = 64
DEPTH = 8
NROW = 1_000_000
PB = 1 << 17
NJ = 8
HP = 32
J_STRIDE = 8 * PB
A_STRIDE = NJ * J_STRIDE
FLAT = 4 * A_STRIDE


def _tc_detile(wt3):
    def body(lo_ref, hi_ref, out_ref):
        evb = lax.bitcast_convert_type(lo_ref[0], jnp.uint32)
        odb = lax.bitcast_convert_type(hi_ref[0], jnp.uint32)
        ev16 = (evb + 0x7FFF + ((evb >> 16) & 1)) >> 16
        od16 = (odb + 0x7FFF + ((odb >> 16) & 1)) >> 16
        packed = lax.bitcast_convert_type(ev16 | (od16 << 16), jnp.float32)
        out_ref[...] = packed.reshape(J_STRIDE // 128, 128)

    return pl.pallas_call(
        body,
        grid=(4, NJ),
        in_specs=[
            pl.BlockSpec((1, 8, PB), lambda a, j: (a, 0, j)),
            pl.BlockSpec((1, 8, PB), lambda a, j: (a + 4, 0, j)),
        ],
        out_specs=pl.BlockSpec(
            (J_STRIDE // 128, 128), lambda a, j: (a * NJ + j, 0)),
        out_shape=jax.ShapeDtypeStruct((FLAT // 128, 128), jnp.float32),
    )(wt3, wt3)


def _build_sc(B):
    bpw = B // NW
    nch = bpw // CHUNK

    mesh = plsc.VectorSubcoreMesh(core_axis_name="c", subcore_axis_name="s")

    @functools.partial(
        pl.kernel,
        out_type=jax.ShapeDtypeStruct((B,), jnp.float32),
        mesh=mesh,
        compiler_params=pltpu.CompilerParams(
            needs_layout_passes=False, use_tc_tiling_on_sc=False),
        scratch_types=[
            pltpu.VMEM((nch, CHUNK), jnp.int32),
            pltpu.VMEM((nch, CHUNK), jnp.int32),
            pltpu.VMEM((HP, CHUNK), jnp.int32),
            pltpu.VMEM((HP, CHUNK), jnp.int32),
            pltpu.VMEM((HP * CHUNK,), jnp.float32),
            pltpu.VMEM((HP * CHUNK,), jnp.float32),
            pltpu.VMEM((bpw,), jnp.float32),
            pltpu.VMEM((bpw,), jnp.float32),
            pltpu.VMEM((bpw,), jnp.float32),
            pltpu.VMEM((LANES,), jnp.float32),
            pltpu.SemaphoreType.DMA,
        ],
    )
    def mf(user_hbm, item_hbm, uw_hbm, iw_hbm, ub_hbm, ib_hbm, bias_hbm,
           out_hbm, uid_v, iid_v, uhx_v, ihx_v, uval_v, ival_v,
           ubr_v, ibr_v, out_v, bias_v, sem):
        wid = lax.axis_index("s") * NC + lax.axis_index("c")
        base = wid * bpw

        pltpu.sync_copy(user_hbm.at[wid], uid_v)
        pltpu.sync_copy(item_hbm.at[wid], iid_v)
        pltpu.sync_copy(bias_hbm, bias_v)

        bias_copies = []
        for c in range(nch):
            sl = pl.ds(c * CHUNK, CHUNK)
            bias_copies.append(pltpu.make_async_copy(
                ub_hbm.at[uid_v.at[c]], ubr_v.at[sl], sem))
            bias_copies.append(pltpu.make_async_copy(
                ib_hbm.at[iid_v.at[c]], ibr_v.at[sl], sem))
        for cp in bias_copies:
            cp.start()

        bias_vec = bias_v[...]

        for c in range(nch):
            def build_body(hp, carry, _c=c):
                coff = (hp >> 3) * A_STRIDE + (hp & 7) * PB
                for g in range(CHUNK // LANES):
                    sl = pl.ds(g * LANES, LANES)
                    r = uid_v[_c, sl]
                    uhx_v[hp, sl] = (
                        (r >> 17) * J_STRIDE + (r & (PB - 1)) + coff)
                    r = iid_v[_c, sl]
                    ihx_v[hp, sl] = (
                        (r >> 17) * J_STRIDE + (r & (PB - 1)) + coff)
                return carry

            lax.fori_loop(0, HP, build_body, 0)

            def fire_one(hp):
                dst = pl.ds(hp * CHUNK, CHUNK)
                pltpu.make_async_copy(
                    uw_hbm.at[uhx_v.at[hp]], uval_v.at[dst], sem).start()
                pltpu.make_async_copy(
                    iw_hbm.at[ihx_v.at[hp]], ival_v.at[dst], sem).start()

            def wait_one(hp):
                dst = pl.ds(hp * CHUNK, CHUNK)
                pltpu.make_async_copy(
                    uw_hbm.at[uhx_v.at[hp]], uval_v.at[dst], sem).wait()
                pltpu.make_async_copy(
                    iw_hbm.at[ihx_v.at[hp]], ival_v.at[dst], sem).wait()

            def fire_body(hp, carry):
                fire_one(hp)

                @pl.when(hp >= DEPTH)
                def _():
                    wait_one(hp - DEPTH)
                return carry

            lax.fori_loop(0, HP, fire_body, 0)

            def drain_body(hp, carry):
                wait_one(hp)
                return carry

            lax.fori_loop(HP - DEPTH, HP, drain_body, 0)
            if c == 0:
                for cp in bias_copies:
                    cp.wait()

            himask = jnp.full((LANES,), 0xFFFF0000, jnp.uint32)

            def unpack2(w):
                bits = plsc.bitcast(w, jnp.uint32)
                ev = plsc.bitcast(bits << 16, jnp.float32)
                od = plsc.bitcast(bits & himask, jnp.float32)
                return ev, od

            for g in range(CHUNK // LANES):
                o = c * CHUNK + g * LANES
                ubv = ubr_v[pl.ds(o, LANES)]
                ibv = ibr_v[pl.ds(o, LANES)]

                def h_body(hp, acc, _g=g):
                    wu = uval_v[pl.ds(hp * CHUNK + _g * LANES, LANES)]
                    wi = ival_v[pl.ds(hp * CHUNK + _g * LANES, LANES)]
                    uev, uod = unpack2(wu)
                    iev, iod = unpack2(wi)
                    return (acc + (uev + ubv) * (iev + ibv)
                            + (uod + ubv) * (iod + ibv))

                acc = lax.fori_loop(
                    0, HP, h_body, jnp.zeros((LANES,), jnp.float32))
                out_v[pl.ds(o, LANES)] = acc + bias_vec

        pltpu.sync_copy(out_v, out_hbm.at[pl.ds(base, bpw)])

    return mf


def kernel(user, item, user_weight, item_weight, user_bias, item_bias, bias):
    B = user.shape[0]
    user_r = user.reshape(NW, B // NW // CHUNK, CHUNK)
    item_r = item.reshape(NW, B // NW // CHUNK, CHUNK)
    uw_flat = _tc_detile(user_weight.T.reshape(8, 8, NROW)).reshape(-1)
    iw_flat = _tc_detile(item_weight.T.reshape(8, 8, NROW)).reshape(-1)
    ub = user_bias.reshape(-1)
    ib = item_bias.reshape(-1)
    bias16 = jnp.broadcast_to(bias, (LANES,)).astype(jnp.float32)
    mf = _build_sc(B)
    return mf(user_r, item_r, uw_flat, iw_flat, ub, ib, bias16)


# speedup vs baseline: 3.2744x; 1.0727x over previous
import functools

import jax
import jax.numpy as jnp
from jax import lax
from jax.experimental import pallas as pl
from jax.experimental.pallas import tpu as pltpu
from jax.experimental.pallas import tpu_sc as plsc

NC = 2
NS = 16
NW = NC * NS
LANES = 16
CHUNK = 128
H = 64
DEPTH = 8
NROW = 1_000_000
PB = 1 << 17
NJ = 8
HP = 32
J_STRIDE = 8 * PB
A_STRIDE = NJ * J_STRIDE
FLAT = 4 * A_STRIDE


def _tc_detile(wt3):
    def body(lo_ref, hi_ref, out_ref):
        evb = lax.bitcast_convert_type(lo_ref[0], jnp.uint32)
        odb = lax.bitcast_convert_type(hi_ref[0], jnp.uint32)
        packed = lax.bitcast_convert_type(
            (evb >> 16) | (odb & jnp.uint32(0xFFFF0000)), jnp.float32)
        out_ref[...] = packed.reshape(J_STRIDE // 128, 128)

    return pl.pallas_call(
        body,
        grid=(4, NJ),
        in_specs=[
            pl.BlockSpec((1, 8, PB), lambda a, j: (a, 0, j)),
            pl.BlockSpec((1, 8, PB), lambda a, j: (a + 4, 0, j)),
        ],
        out_specs=pl.BlockSpec(
            (J_STRIDE // 128, 128), lambda a, j: (a * NJ + j, 0)),
        out_shape=jax.ShapeDtypeStruct((FLAT // 128, 128), jnp.float32),
    )(wt3, wt3)


def _build_sc(B):
    bpw = B // NW
    nch = bpw // CHUNK

    mesh = plsc.VectorSubcoreMesh(core_axis_name="c", subcore_axis_name="s")

    @functools.partial(
        pl.kernel,
        out_type=jax.ShapeDtypeStruct((B,), jnp.float32),
        mesh=mesh,
        compiler_params=pltpu.CompilerParams(
            needs_layout_passes=False, use_tc_tiling_on_sc=False),
        scratch_types=[
            pltpu.VMEM((nch, CHUNK), jnp.int32),
            pltpu.VMEM((nch, CHUNK), jnp.int32),
            pltpu.VMEM((HP, CHUNK), jnp.int32),
            pltpu.VMEM((HP, CHUNK), jnp.int32),
            pltpu.VMEM((HP * CHUNK,), jnp.float32),
            pltpu.VMEM((HP * CHUNK,), jnp.float32),
            pltpu.VMEM((bpw,), jnp.float32),
            pltpu.VMEM((bpw,), jnp.float32),
            pltpu.VMEM((bpw,), jnp.float32),
            pltpu.VMEM((LANES,), jnp.float32),
            pltpu.SemaphoreType.DMA,
        ],
    )
    def mf(user_hbm, item_hbm, uw_hbm, iw_hbm, ub_hbm, ib_hbm, bias_hbm,
           out_hbm, uid_v, iid_v, uhx_v, ihx_v, uval_v, ival_v,
           ubr_v, ibr_v, out_v, bias_v, sem):
        wid = lax.axis_index("s") * NC + lax.axis_index("c")
        base = wid * bpw

        pltpu.sync_copy(user_hbm.at[wid], uid_v)
        pltpu.sync_copy(item_hbm.at[wid], iid_v)
        pltpu.sync_copy(bias_hbm, bias_v)

        bias_copies = []
        for c in range(nch):
            sl = pl.ds(c * CHUNK, CHUNK)
            bias_copies.append(pltpu.make_async_copy(
                ub_hbm.at[uid_v.at[c]], ubr_v.at[sl], sem))
            bias_copies.append(pltpu.make_async_copy(
                ib_hbm.at[iid_v.at[c]], ibr_v.at[sl], sem))
        for cp in bias_copies:
            cp.start()

        bias_vec = bias_v[...]

        for c in range(nch):
            def build_body(hp, carry, _c=c):
                coff = (hp >> 3) * A_STRIDE + (hp & 7) * PB
                for g in range(CHUNK // LANES):
                    sl = pl.ds(g * LANES, LANES)
                    r = uid_v[_c, sl]
                    uhx_v[hp, sl] = (
                        (r >> 17) * J_STRIDE + (r & (PB - 1)) + coff)
                    r = iid_v[_c, sl]
                    ihx_v[hp, sl] = (
                        (r >> 17) * J_STRIDE + (r & (PB - 1)) + coff)
                return carry

            lax.fori_loop(0, HP, build_body, 0)

            def fire_one(hp):
                dst = pl.ds(hp * CHUNK, CHUNK)
                pltpu.make_async_copy(
                    uw_hbm.at[uhx_v.at[hp]], uval_v.at[dst], sem).start()
                pltpu.make_async_copy(
                    iw_hbm.at[ihx_v.at[hp]], ival_v.at[dst], sem).start()

            def wait_one(hp):
                dst = pl.ds(hp * CHUNK, CHUNK)
                pltpu.make_async_copy(
                    uw_hbm.at[uhx_v.at[hp]], uval_v.at[dst], sem).wait()
                pltpu.make_async_copy(
                    iw_hbm.at[ihx_v.at[hp]], ival_v.at[dst], sem).wait()

            def fire_body(hp, carry):
                fire_one(hp)

                @pl.when(hp >= DEPTH)
                def _():
                    wait_one(hp - DEPTH)
                return carry

            lax.fori_loop(0, HP, fire_body, 0)

            def drain_body(hp, carry):
                wait_one(hp)
                return carry

            lax.fori_loop(HP - DEPTH, HP, drain_body, 0)
            if c == 0:
                for cp in bias_copies:
                    cp.wait()

            himask = jnp.full((LANES,), 0xFFFF0000, jnp.uint32)

            def unpack2(w):
                bits = plsc.bitcast(w, jnp.uint32)
                ev = plsc.bitcast(bits << 16, jnp.float32)
                od = plsc.bitcast(bits & himask, jnp.float32)
                return ev, od

            for g in range(CHUNK // LANES):
                o = c * CHUNK + g * LANES
                ubv = ubr_v[pl.ds(o, LANES)]
                ibv = ibr_v[pl.ds(o, LANES)]

                def h_body(hp, acc, _g=g):
                    wu = uval_v[pl.ds(hp * CHUNK + _g * LANES, LANES)]
                    wi = ival_v[pl.ds(hp * CHUNK + _g * LANES, LANES)]
                    uev, uod = unpack2(wu)
                    iev, iod = unpack2(wi)
                    return (acc + (uev + ubv) * (iev + ibv)
                            + (uod + ubv) * (iod + ibv))

                acc = lax.fori_loop(
                    0, HP, h_body, jnp.zeros((LANES,), jnp.float32))
                out_v[pl.ds(o, LANES)] = acc + bias_vec

        pltpu.sync_copy(out_v, out_hbm.at[pl.ds(base, bpw)])

    return mf


def kernel(user, item, user_weight, item_weight, user_bias, item_bias, bias):
    B = user.shape[0]
    user_r = user.reshape(NW, B // NW // CHUNK, CHUNK)
    item_r = item.reshape(NW, B // NW // CHUNK, CHUNK)
    uw_flat = _tc_detile(user_weight.T.reshape(8, 8, NROW)).reshape(-1)
    iw_flat = _tc_detile(item_weight.T.reshape(8, 8, NROW)).reshape(-1)
    ub = user_bias.reshape(-1)
    ib = item_bias.reshape(-1)
    bias16 = jnp.broadcast_to(bias, (LANES,)).astype(jnp.float32)
    mf = _build_sc(B)
    return mf(user_r, item_r, uw_flat, iw_flat, ub, ib, bias16)
